# Initial kernel scaffold; baseline (speedup 1.0000x reference)
import jax
import jax.numpy as jnp
from jax.experimental import pallas as pl


def kernel(x, params, edge_src, edge_dst, edge_w):
    raise NotImplementedError("write your pallas kernel here")


# speedup vs baseline: 28.0866x; 28.0866x over previous
import functools
import math

import jax
import jax.numpy as jnp
from jax import lax
from jax.experimental import pallas as pl
from jax.experimental.pallas import tpu as pltpu
from jax.experimental.pallas import tpu_sc as plsc

_KS = 3
_NS = [800, 1600, 3200, 6400, 12800, 25600]
_LVLS = ["l0", "l1", "l2", "l3", "l4", "l5"]
_NW = 32
_F32 = jnp.float32
_I32 = jnp.int32


def _mesh():
    return plsc.VectorSubcoreMesh(core_axis_name="c", subcore_axis_name="s")


_SC_PARAMS = pltpu.CompilerParams(
    needs_layout_passes=False, use_tc_tiling_on_sc=False)


def _wid():
    return lax.axis_index("s") * 2 + lax.axis_index("c")


def _round_up(x, m):
    return (x + m - 1) // m * m


@functools.lru_cache(maxsize=None)
def _make_deg(n):
    e = n * 16
    npad = _round_up(n, 256)
    epw = e // _NW
    ce = min(epw, 1600)
    nch = epw // ce

    @functools.partial(
        pl.kernel,
        out_type=jax.ShapeDtypeStruct((_NW, npad), _F32),
        mesh=_mesh(),
        compiler_params=_SC_PARAMS,
        scratch_types=[
            pltpu.VMEM((npad,), _F32),
            pltpu.VMEM((ce,), _I32),
            pltpu.VMEM((ce,), _F32),
        ],
    )
    def deg_kernel(pk_hbm, w_hbm, out_hbm, part_v, pk_v, w_v):
        w = _wid()
        zero16 = jnp.zeros((16,), _F32)

        def zero_body(i, _):
            part_v[pl.ds(i * 16, 16)] = zero16
            return 0

        lax.fori_loop(0, npad // 16, zero_body, 0)
        ebase = w * epw

        def chunk_body(ch, _):
            off = ebase + ch * ce
            pltpu.sync_copy(pk_hbm.at[pl.ds(off, ce)], pk_v)
            pltpu.sync_copy(w_hbm.at[pl.ds(off, ce)], w_v)

            def grp(i, _):
                pk16 = pk_v[pl.ds(i * 16, 16)]
                d16 = lax.shift_right_logical(pk16, 16)
                w16 = w_v[pl.ds(i * 16, 16)]
                plsc.addupdate_scatter(part_v, [d16], w16)
                return 0

            lax.fori_loop(0, ce // 16, grp, 0)
            return 0

        lax.fori_loop(0, nch, chunk_body, 0)
        pltpu.sync_copy(part_v, out_hbm.at[w])

    return deg_kernel


@functools.lru_cache(maxsize=None)
def _make_isd(npad):
    def body(parts_ref, out_ref):
        deg = jnp.sum(parts_ref[...], axis=0, keepdims=True) + 1e-6
        out_ref[...] = lax.rsqrt(deg)

    return pl.pallas_call(
        body,
        out_shape=jax.ShapeDtypeStruct((1, npad), _F32),
    )


@functools.lru_cache(maxsize=None)
def _make_wn(n):
    e = n * 16
    npad = _round_up(n, 256)
    epw = e // _NW
    ce = min(epw, 1600)
    nch = epw // ce

    @functools.partial(
        pl.kernel,
        out_type=jax.ShapeDtypeStruct((e,), _F32),
        mesh=_mesh(),
        compiler_params=_SC_PARAMS,
        scratch_types=[
            pltpu.VMEM((npad,), _F32),
            pltpu.VMEM((ce,), _I32),
            pltpu.VMEM((ce,), _F32),
            pltpu.VMEM((ce,), _F32),
        ],
    )
    def wn_kernel(pk_hbm, w_hbm, isd_hbm, out_hbm, isd_v, pk_v, w_v, o_v):
        w = _wid()
        pltpu.sync_copy(isd_hbm.at[0], isd_v)
        ebase = w * epw

        def chunk_body(ch, _):
            off = ebase + ch * ce
            pltpu.sync_copy(pk_hbm.at[pl.ds(off, ce)], pk_v)
            pltpu.sync_copy(w_hbm.at[pl.ds(off, ce)], w_v)

            def grp(i, _):
                pk16 = pk_v[pl.ds(i * 16, 16)]
                s16 = jnp.bitwise_and(pk16, 0xFFFF)
                d16 = lax.shift_right_logical(pk16, 16)
                w16 = w_v[pl.ds(i * 16, 16)]
                a = plsc.load_gather(isd_v, [s16])
                b = plsc.load_gather(isd_v, [d16])
                o_v[pl.ds(i * 16, 16)] = w16 * a * b
                return 0

            lax.fori_loop(0, ce // 16, grp, 0)
            pltpu.sync_copy(o_v, out_hbm.at[pl.ds(off, ce)])
            return 0

        lax.fori_loop(0, nch, chunk_body, 0)

    return wn_kernel


@functools.lru_cache(maxsize=None)
def _make_apply(n, c_cols, cheb):
    e = n * 16
    ce = 1600
    nch = e // ce
    budget = 118000
    ncol = max(1, min(budget // (2 * n), 32, -(-c_cols // _NW)))
    nsweep = -(-c_cols // (_NW * ncol))
    ob = min(n, 1600)
    nob = n // ob

    scratch = [
        pltpu.VMEM((ncol * n,), _F32),
        pltpu.VMEM((ncol * n,), _F32),
        pltpu.VMEM((ce,), _I32),
        pltpu.VMEM((ce,), _F32),
    ]
    if cheb:
        scratch += [pltpu.VMEM((ob,), _F32), pltpu.VMEM((ob,), _F32)]

    def body(x_hbm, pk_hbm, wn_hbm, *rest):
        if cheb:
            prev_hbm, out_hbm, x_v, a_v, pk_v, wn_v, p_v, o_v = rest
        else:
            out_hbm, x_v, a_v, pk_v, wn_v = rest
        w = _wid()

        for sw in range(nsweep):
            base = (sw * _NW + w) * ncol

            for j in range(ncol):
                col = base + j

                @pl.when(col < c_cols)
                def _():
                    pltpu.sync_copy(x_hbm.at[col], x_v.at[pl.ds(j * n, n)])
                    pltpu.sync_copy(x_hbm.at[col], a_v.at[pl.ds(j * n, n)])

            @pl.when(base < c_cols)
            def _():
                def chunk_body(ch, _):
                    off = ch * ce
                    pltpu.sync_copy(pk_hbm.at[pl.ds(off, ce)], pk_v)
                    pltpu.sync_copy(wn_hbm.at[pl.ds(off, ce)], wn_v)

                    def grp(i, _):
                        pk16 = pk_v[pl.ds(i * 16, 16)]
                        s16 = jnp.bitwise_and(pk16, 0xFFFF)
                        d16 = lax.shift_right_logical(pk16, 16)
                        w16 = wn_v[pl.ds(i * 16, 16)]
                        for j in range(ncol):
                            v = plsc.load_gather(x_v, [s16 + j * n])
                            plsc.addupdate_scatter(a_v, [d16 + j * n], -(v * w16))
                        return 0

                    lax.fori_loop(0, ce // 16, grp, 0)
                    return 0

                lax.fori_loop(0, nch, chunk_body, 0)

            for j in range(ncol):
                col = base + j

                @pl.when(col < c_cols)
                def _():
                    if not cheb:
                        pltpu.sync_copy(a_v.at[pl.ds(j * n, n)], out_hbm.at[col])
                    else:
                        def wb(o, _):
                            pltpu.sync_copy(prev_hbm.at[col, pl.ds(o * ob, ob)], p_v)

                            def inner(i, _):
                                av = a_v[pl.ds(j * n + o * ob + i * 16, 16)]
                                pv = p_v[pl.ds(i * 16, 16)]
                                o_v[pl.ds(i * 16, 16)] = 2.0 * av - pv
                                return 0

                            lax.fori_loop(0, ob // 16, inner, 0)
                            pltpu.sync_copy(o_v, out_hbm.at[col, pl.ds(o * ob, ob)])
                            return 0

                        lax.fori_loop(0, nob, wb, 0)

    return functools.partial(
        pl.kernel,
        out_type=jax.ShapeDtypeStruct((c_cols, n), _F32),
        mesh=_mesh(),
        compiler_params=_SC_PARAMS,
        scratch_types=scratch,
    )(body)


@functools.lru_cache(maxsize=None)
def _make_mm(f, m, n, b, has_bias, f2, act):
    nb = min(1024, n)
    grid = (b, -(-n // nb))

    def body(*refs):
        idx = 0
        w_ref = refs[idx]; idx += 1
        t_ref = refs[idx]; idx += 1
        if has_bias:
            bias_ref = refs[idx]; idx += 1
        if f2:
            w2_ref = refs[idx]; idx += 1
            t2_ref = refs[idx]; idx += 1
        out_ref = refs[idx]
        y = lax.dot_general(
            w_ref[...], t_ref[0],
            (((0,), (0,)), ((), ())),
            precision=lax.Precision.HIGHEST,
            preferred_element_type=_F32,
        )
        if f2:
            y = y + lax.dot_general(
                w2_ref[...], t2_ref[0],
                (((0,), (0,)), ((), ())),
                precision=lax.Precision.HIGHEST,
                preferred_element_type=_F32,
            )
        if has_bias:
            y = y + bias_ref[...]
        if act == "relu":
            y = jnp.maximum(y, 0.0)
        elif act == "lsm":
            y = y - jnp.max(y, axis=0, keepdims=True)
            y = y - jnp.log(jnp.sum(jnp.exp(y), axis=0, keepdims=True))
        out_ref[0] = y

    in_specs = [
        pl.BlockSpec((f, m), lambda bi, ni: (0, 0)),
        pl.BlockSpec((1, f, nb), lambda bi, ni: (bi, 0, ni)),
    ]
    if has_bias:
        in_specs.append(pl.BlockSpec((m, 1), lambda bi, ni: (0, 0)))
    if f2:
        in_specs.append(pl.BlockSpec((f2, m), lambda bi, ni: (0, 0)))
        in_specs.append(pl.BlockSpec((1, f2, nb), lambda bi, ni: (bi, 0, ni)))

    return pl.pallas_call(
        body,
        grid=grid,
        in_specs=in_specs,
        out_specs=pl.BlockSpec((1, m, nb), lambda bi, ni: (bi, 0, ni)),
        out_shape=jax.ShapeDtypeStruct((b, m, n), _F32),
    )


@functools.lru_cache(maxsize=None)
def _make_max(r, ncols):
    br = min(r, 256)
    bn = min(ncols, 2048)
    grid = (-(-r // br), -(-ncols // bn))

    def body(a_ref, b_ref, o_ref):
        o_ref[...] = jnp.maximum(a_ref[...], b_ref[...])

    spec = pl.BlockSpec((br, bn), lambda i, j: (i, j))
    return pl.pallas_call(
        body,
        grid=grid,
        in_specs=[spec, spec],
        out_specs=spec,
        out_shape=jax.ShapeDtypeStruct((r, ncols), _F32),
    )


def _cheb_T(xbc, graph):
    pk, wn, n = graph
    b, cin, _ = xbc.shape
    c = b * cin
    x2 = xbc.reshape(c, n)
    t1 = _make_apply(n, c, False)(x2, pk, wn)
    t2 = _make_apply(n, c, True)(t1, pk, wn, x2)
    return jnp.concatenate(
        [xbc, t1.reshape(b, cin, n), t2.reshape(b, cin, n)], axis=1)


def _conv_k3(xbc, p, graph, act):
    t = _cheb_T(xbc, graph)
    b, f, n = t.shape
    m = p["W"].shape[2]
    wf = p["W"].reshape(f, m)
    bias = p["b"].reshape(m, 1)
    return _make_mm(f, m, n, b, True, 0, act)(wf, t, bias)


def _res_block(xbc, p, graph):
    h = _conv_k3(xbc, p["conv1"], graph, "relu")
    t = _cheb_T(h, graph)
    b, f, n = t.shape
    cin = xbc.shape[1]
    m = p["conv2"]["W"].shape[2]
    w2f = p["conv2"]["W"].reshape(f, m)
    bias = p["conv2"]["b"].reshape(m, 1)
    wscf = p["sc"]["W"].reshape(cin, m)
    return _make_mm(f, m, n, b, True, cin, "relu")(w2f, t, bias, wscf, xbc)


def _pool(t):
    b, c, n = t.shape
    a = t[:, :, 0::2].reshape(b * c, n // 2)
    bb = t[:, :, 1::2].reshape(b * c, n // 2)
    return _make_max(b * c, n // 2)(a, bb).reshape(b, c, n // 2)


def _unpool(t):
    return jnp.repeat(t, 2, axis=2)


def kernel(x, params, edge_src, edge_dst, edge_w):
    graphs = {}
    for i, lvl in enumerate(_LVLS):
        n = _NS[i]
        src = edge_src[lvl].astype(_I32)
        dst = edge_dst[lvl].astype(_I32)
        pk = jnp.bitwise_or(src, dst << 16)
        ew = edge_w[lvl].astype(_F32)
        parts = _make_deg(n)(pk, ew)
        isd = _make_isd(_round_up(n, 256))(parts)
        wn = _make_wn(n)(pk, ew, isd)
        graphs[lvl] = (pk, wn, n)

    h = _conv_k3(x, params["enc_conv"], graphs["l5"], "relu")
    e5 = _res_block(h, params["enc_b5"], graphs["l5"])
    e4 = _res_block(_pool(e5), params["enc_b4"], graphs["l4"])
    e3 = _res_block(_pool(e4), params["enc_b3"], graphs["l3"])
    e2 = _res_block(_pool(e3), params["enc_b2"], graphs["l2"])
    e1 = _res_block(_pool(e2), params["enc_b1"], graphs["l1"])
    e0 = _res_block(_pool(e1), params["enc_b0"], graphs["l0"])
    d1 = _res_block(jnp.concatenate([_unpool(e0), e1], axis=1),
                    params["dec_b1"], graphs["l1"])
    d2 = _res_block(jnp.concatenate([_unpool(d1), e2], axis=1),
                    params["dec_b2"], graphs["l2"])
    d3 = _res_block(jnp.concatenate([_unpool(d2), e3], axis=1),
                    params["dec_b3"], graphs["l3"])
    d4 = _res_block(jnp.concatenate([_unpool(d3), e4], axis=1),
                    params["dec_b4"], graphs["l4"])
    d5 = _res_block(jnp.concatenate([_unpool(d4), e5], axis=1),
                    params["dec_b5"], graphs["l5"])

    b, cin, n = d5.shape
    wdec = params["dec_conv"]["W"].reshape(cin, 10)
    return _make_mm(cin, 10, n, b, False, 0, "lsm")(wdec, d5)


# speedup vs baseline: 36.9597x; 1.3159x over previous
import functools
import math

import jax
import jax.numpy as jnp
from jax import lax
from jax.experimental import pallas as pl
from jax.experimental.pallas import tpu as pltpu
from jax.experimental.pallas import tpu_sc as plsc

_KS = 3
_NS = [800, 1600, 3200, 6400, 12800, 25600]
_LVLS = ["l0", "l1", "l2", "l3", "l4", "l5"]
_NW = 32
_F32 = jnp.float32
_I32 = jnp.int32


def _mesh():
    return plsc.VectorSubcoreMesh(core_axis_name="c", subcore_axis_name="s")


_SC_PARAMS = pltpu.CompilerParams(
    needs_layout_passes=False, use_tc_tiling_on_sc=False)


def _wid():
    return lax.axis_index("s") * 2 + lax.axis_index("c")


def _round_up(x, m):
    return (x + m - 1) // m * m


@functools.lru_cache(maxsize=None)
def _make_deg(n):
    e = n * 16
    npad = _round_up(n, 256)
    epw = e // _NW
    ce = min(epw, 1600)
    nch = epw // ce

    @functools.partial(
        pl.kernel,
        out_type=jax.ShapeDtypeStruct((_NW, npad), _F32),
        mesh=_mesh(),
        compiler_params=_SC_PARAMS,
        scratch_types=[
            pltpu.VMEM((npad,), _F32),
            pltpu.VMEM((ce,), _I32),
            pltpu.VMEM((ce,), _F32),
        ],
    )
    def deg_kernel(pk_hbm, w_hbm, out_hbm, part_v, pk_v, w_v):
        w = _wid()
        zero16 = jnp.zeros((16,), _F32)

        def zero_body(i, _):
            part_v[pl.ds(i * 16, 16)] = zero16
            return 0

        lax.fori_loop(0, npad // 16, zero_body, 0)
        ebase = w * epw

        def chunk_body(ch, _):
            off = ebase + ch * ce
            pltpu.sync_copy(pk_hbm.at[pl.ds(off, ce)], pk_v)
            pltpu.sync_copy(w_hbm.at[pl.ds(off, ce)], w_v)

            def grp(i, _):
                pk16 = pk_v[pl.ds(i * 16, 16)]
                d16 = lax.shift_right_logical(pk16, 16)
                w16 = w_v[pl.ds(i * 16, 16)]
                plsc.addupdate_scatter(part_v, [d16], w16)
                return 0

            lax.fori_loop(0, ce // 16, grp, 0)
            return 0

        lax.fori_loop(0, nch, chunk_body, 0)
        pltpu.sync_copy(part_v, out_hbm.at[w])

    return deg_kernel


@functools.lru_cache(maxsize=None)
def _make_isd(npad):
    def body(parts_ref, out_ref):
        deg = jnp.sum(parts_ref[...], axis=0, keepdims=True) + 1e-6
        out_ref[...] = lax.rsqrt(deg)

    return pl.pallas_call(
        body,
        out_shape=jax.ShapeDtypeStruct((1, npad), _F32),
    )


@functools.lru_cache(maxsize=None)
def _make_wn(n):
    e = n * 16
    npad = _round_up(n, 256)
    epw = e // _NW
    ce = min(epw, 1600)
    nch = epw // ce

    @functools.partial(
        pl.kernel,
        out_type=jax.ShapeDtypeStruct((e,), _F32),
        mesh=_mesh(),
        compiler_params=_SC_PARAMS,
        scratch_types=[
            pltpu.VMEM((npad,), _F32),
            pltpu.VMEM((ce,), _I32),
            pltpu.VMEM((ce,), _F32),
            pltpu.VMEM((ce,), _F32),
        ],
    )
    def wn_kernel(pk_hbm, w_hbm, isd_hbm, out_hbm, isd_v, pk_v, w_v, o_v):
        w = _wid()
        pltpu.sync_copy(isd_hbm.at[0], isd_v)
        ebase = w * epw

        def chunk_body(ch, _):
            off = ebase + ch * ce
            pltpu.sync_copy(pk_hbm.at[pl.ds(off, ce)], pk_v)
            pltpu.sync_copy(w_hbm.at[pl.ds(off, ce)], w_v)

            def grp(i, _):
                pk16 = pk_v[pl.ds(i * 16, 16)]
                s16 = jnp.bitwise_and(pk16, 0xFFFF)
                d16 = lax.shift_right_logical(pk16, 16)
                w16 = w_v[pl.ds(i * 16, 16)]
                a = plsc.load_gather(isd_v, [s16])
                b = plsc.load_gather(isd_v, [d16])
                o_v[pl.ds(i * 16, 16)] = w16 * a * b
                return 0

            lax.fori_loop(0, ce // 16, grp, 0)
            pltpu.sync_copy(o_v, out_hbm.at[pl.ds(off, ce)])
            return 0

        lax.fori_loop(0, nch, chunk_body, 0)

    return wn_kernel


@functools.lru_cache(maxsize=None)
def _make_apply(n, c_cols, cheb):
    e = n * 16
    ce = 3200
    nch = e // ce
    half = nch // 2
    budget = 112000
    ncol = max(1, min(budget // (2 * n), 32, -(-c_cols // _NW)))
    nsweep = -(-c_cols // (_NW * ncol))

    scratch = [
        pltpu.VMEM((ncol * n,), _F32),
        pltpu.VMEM((ncol * n,), _F32),
        pltpu.VMEM((2 * ce,), _I32),
        pltpu.VMEM((2 * ce,), _I32),
        pltpu.SemaphoreType.DMA,
        pltpu.SemaphoreType.DMA,
    ]

    def body(x_hbm, ew_hbm, *rest):
        if cheb:
            prev_hbm, out_hbm, x_v, a_v, e0_v, e1_v, sem0, sem1 = rest
        else:
            out_hbm, x_v, a_v, e0_v, e1_v, sem0, sem1 = rest
        w = _wid()

        def compute(buf):
            def grp(i, _):
                pk16 = buf[pl.ds(i * 32, 16)]
                s16 = jnp.bitwise_and(pk16, 0xFFFF)
                d16 = lax.shift_right_logical(pk16, 16)
                w16 = plsc.bitcast(buf[pl.ds(i * 32 + 16, 16)], _F32)
                for j in range(ncol):
                    v = plsc.load_gather(x_v, [s16 + j * n])
                    plsc.addupdate_scatter(a_v, [d16 + j * n], -(v * w16))
                return 0

            lax.fori_loop(0, ce // 16, grp, 0)

        for sw in range(nsweep):
            base = (sw * _NW + w) * ncol

            for j in range(ncol):
                col = base + j

                @pl.when(col < c_cols)
                def _():
                    pltpu.sync_copy(x_hbm.at[col], x_v.at[pl.ds(j * n, n)])
                    pltpu.sync_copy(x_hbm.at[col], a_v.at[pl.ds(j * n, n)])

            @pl.when(base < c_cols)
            def _():
                pltpu.async_copy(ew_hbm.at[pl.ds(0, 2 * ce)], e0_v, sem0)

                def pair(p, _):
                    c0 = 2 * p
                    pltpu.async_copy(
                        ew_hbm.at[pl.ds((c0 + 1) * 2 * ce, 2 * ce)], e1_v, sem1)
                    pltpu.make_async_copy(
                        ew_hbm.at[pl.ds(c0 * 2 * ce, 2 * ce)], e0_v, sem0).wait()
                    compute(e0_v)

                    @pl.when(c0 + 2 < nch)
                    def _():
                        pltpu.async_copy(
                            ew_hbm.at[pl.ds((c0 + 2) * 2 * ce, 2 * ce)],
                            e0_v, sem0)

                    pltpu.make_async_copy(
                        ew_hbm.at[pl.ds((c0 + 1) * 2 * ce, 2 * ce)],
                        e1_v, sem1).wait()
                    compute(e1_v)
                    return 0

                lax.fori_loop(0, half, pair, 0)

            for j in range(ncol):
                col = base + j

                @pl.when(col < c_cols)
                def _():
                    if not cheb:
                        pltpu.sync_copy(a_v.at[pl.ds(j * n, n)], out_hbm.at[col])
                    else:
                        pltpu.sync_copy(prev_hbm.at[col], x_v.at[pl.ds(j * n, n)])

                        def inner(i, _):
                            av = a_v[pl.ds(j * n + i * 16, 16)]
                            pv = x_v[pl.ds(j * n + i * 16, 16)]
                            x_v[pl.ds(j * n + i * 16, 16)] = 2.0 * av - pv
                            return 0

                        lax.fori_loop(0, n // 16, inner, 0)
                        pltpu.sync_copy(x_v.at[pl.ds(j * n, n)], out_hbm.at[col])

    return functools.partial(
        pl.kernel,
        out_type=jax.ShapeDtypeStruct((c_cols, n), _F32),
        mesh=_mesh(),
        compiler_params=_SC_PARAMS,
        scratch_types=scratch,
    )(body)


@functools.lru_cache(maxsize=None)
def _make_mm(f, m, n, b, has_bias, f2, act):
    nb = min(1024, n)
    grid = (b, -(-n // nb))

    def body(*refs):
        idx = 0
        w_ref = refs[idx]; idx += 1
        t_ref = refs[idx]; idx += 1
        if has_bias:
            bias_ref = refs[idx]; idx += 1
        if f2:
            w2_ref = refs[idx]; idx += 1
            t2_ref = refs[idx]; idx += 1
        out_ref = refs[idx]
        y = lax.dot_general(
            w_ref[...], t_ref[0],
            (((0,), (0,)), ((), ())),
            precision=lax.Precision.HIGHEST,
            preferred_element_type=_F32,
        )
        if f2:
            y = y + lax.dot_general(
                w2_ref[...], t2_ref[0],
                (((0,), (0,)), ((), ())),
                precision=lax.Precision.HIGHEST,
                preferred_element_type=_F32,
            )
        if has_bias:
            y = y + bias_ref[...]
        if act == "relu":
            y = jnp.maximum(y, 0.0)
        elif act == "lsm":
            y = y - jnp.max(y, axis=0, keepdims=True)
            y = y - jnp.log(jnp.sum(jnp.exp(y), axis=0, keepdims=True))
        out_ref[0] = y

    in_specs = [
        pl.BlockSpec((f, m), lambda bi, ni: (0, 0)),
        pl.BlockSpec((1, f, nb), lambda bi, ni: (bi, 0, ni)),
    ]
    if has_bias:
        in_specs.append(pl.BlockSpec((m, 1), lambda bi, ni: (0, 0)))
    if f2:
        in_specs.append(pl.BlockSpec((f2, m), lambda bi, ni: (0, 0)))
        in_specs.append(pl.BlockSpec((1, f2, nb), lambda bi, ni: (bi, 0, ni)))

    return pl.pallas_call(
        body,
        grid=grid,
        in_specs=in_specs,
        out_specs=pl.BlockSpec((1, m, nb), lambda bi, ni: (bi, 0, ni)),
        out_shape=jax.ShapeDtypeStruct((b, m, n), _F32),
    )


@functools.lru_cache(maxsize=None)
def _make_max(r, ncols):
    br = min(r, 256)
    bn = min(ncols, 2048)
    grid = (-(-r // br), -(-ncols // bn))

    def body(a_ref, b_ref, o_ref):
        o_ref[...] = jnp.maximum(a_ref[...], b_ref[...])

    spec = pl.BlockSpec((br, bn), lambda i, j: (i, j))
    return pl.pallas_call(
        body,
        grid=grid,
        in_specs=[spec, spec],
        out_specs=spec,
        out_shape=jax.ShapeDtypeStruct((r, ncols), _F32),
    )


def _cheb_T(xbc, graph):
    ew, n = graph
    b, cin, _ = xbc.shape
    c = b * cin
    x2 = xbc.reshape(c, n)
    t1 = _make_apply(n, c, False)(x2, ew)
    t2 = _make_apply(n, c, True)(t1, ew, x2)
    return jnp.concatenate(
        [xbc, t1.reshape(b, cin, n), t2.reshape(b, cin, n)], axis=1)


def _conv_k3(xbc, p, graph, act):
    t = _cheb_T(xbc, graph)
    b, f, n = t.shape
    m = p["W"].shape[2]
    wf = p["W"].reshape(f, m)
    bias = p["b"].reshape(m, 1)
    return _make_mm(f, m, n, b, True, 0, act)(wf, t, bias)


def _res_block(xbc, p, graph):
    h = _conv_k3(xbc, p["conv1"], graph, "relu")
    t = _cheb_T(h, graph)
    b, f, n = t.shape
    cin = xbc.shape[1]
    m = p["conv2"]["W"].shape[2]
    w2f = p["conv2"]["W"].reshape(f, m)
    bias = p["conv2"]["b"].reshape(m, 1)
    wscf = p["sc"]["W"].reshape(cin, m)
    return _make_mm(f, m, n, b, True, cin, "relu")(w2f, t, bias, wscf, xbc)


def _pool(t):
    b, c, n = t.shape
    a = t[:, :, 0::2].reshape(b * c, n // 2)
    bb = t[:, :, 1::2].reshape(b * c, n // 2)
    return _make_max(b * c, n // 2)(a, bb).reshape(b, c, n // 2)


def _unpool(t):
    return jnp.repeat(t, 2, axis=2)


def kernel(x, params, edge_src, edge_dst, edge_w):
    graphs = {}
    for i, lvl in enumerate(_LVLS):
        n = _NS[i]
        src = edge_src[lvl].astype(_I32)
        dst = edge_dst[lvl].astype(_I32)
        pk = jnp.bitwise_or(src, dst << 16)
        ew = edge_w[lvl].astype(_F32)
        parts = _make_deg(n)(pk, ew)
        isd = _make_isd(_round_up(n, 256))(parts)
        wn = _make_wn(n)(pk, ew, isd)
        wn_i = lax.bitcast_convert_type(wn, _I32)
        epk = jnp.stack(
            [pk.reshape(-1, 16), wn_i.reshape(-1, 16)], axis=1).reshape(-1)
        graphs[lvl] = (epk, n)

    h = _conv_k3(x, params["enc_conv"], graphs["l5"], "relu")
    e5 = _res_block(h, params["enc_b5"], graphs["l5"])
    e4 = _res_block(_pool(e5), params["enc_b4"], graphs["l4"])
    e3 = _res_block(_pool(e4), params["enc_b3"], graphs["l3"])
    e2 = _res_block(_pool(e3), params["enc_b2"], graphs["l2"])
    e1 = _res_block(_pool(e2), params["enc_b1"], graphs["l1"])
    e0 = _res_block(_pool(e1), params["enc_b0"], graphs["l0"])
    d1 = _res_block(jnp.concatenate([_unpool(e0), e1], axis=1),
                    params["dec_b1"], graphs["l1"])
    d2 = _res_block(jnp.concatenate([_unpool(d1), e2], axis=1),
                    params["dec_b2"], graphs["l2"])
    d3 = _res_block(jnp.concatenate([_unpool(d2), e3], axis=1),
                    params["dec_b3"], graphs["l3"])
    d4 = _res_block(jnp.concatenate([_unpool(d3), e4], axis=1),
                    params["dec_b4"], graphs["l4"])
    d5 = _res_block(jnp.concatenate([_unpool(d4), e5], axis=1),
                    params["dec_b5"], graphs["l5"])

    b, cin, n = d5.shape
    wdec = params["dec_conv"]["W"].reshape(cin, 10)
    return _make_mm(cin, 10, n, b, False, 0, "lsm")(wdec, d5)


# speedup vs baseline: 83.0229x; 2.2463x over previous
import functools
import math

import jax
import jax.numpy as jnp
from jax import lax
from jax.experimental import pallas as pl
from jax.experimental.pallas import tpu as pltpu
from jax.experimental.pallas import tpu_sc as plsc

_KS = 3
_NS = [800, 1600, 3200, 6400, 12800, 25600]
_LVLS = ["l0", "l1", "l2", "l3", "l4", "l5"]
_NW = 32
_F32 = jnp.float32
_I32 = jnp.int32


def _mesh():
    return plsc.VectorSubcoreMesh(core_axis_name="c", subcore_axis_name="s")


_SC_PARAMS = pltpu.CompilerParams(
    needs_layout_passes=False, use_tc_tiling_on_sc=False)


def _wid():
    return lax.axis_index("s") * 2 + lax.axis_index("c")


def _round_up(x, m):
    return (x + m - 1) // m * m


@functools.lru_cache(maxsize=None)
def _make_deg(n):
    e = n * 16
    npad = _round_up(n, 256)
    epw = e // _NW
    ce = min(epw, 1600)
    nch = epw // ce

    @functools.partial(
        pl.kernel,
        out_type=jax.ShapeDtypeStruct((_NW, npad), _F32),
        mesh=_mesh(),
        compiler_params=_SC_PARAMS,
        scratch_types=[
            pltpu.VMEM((npad,), _F32),
            pltpu.VMEM((ce,), _I32),
            pltpu.VMEM((ce,), _F32),
        ],
    )
    def deg_kernel(pk_hbm, w_hbm, out_hbm, part_v, pk_v, w_v):
        w = _wid()
        zero16 = jnp.zeros((16,), _F32)

        @plsc.parallel_loop(0, npad // 16, unroll=8)
        def _(i):
            part_v[pl.ds(i * 16, 16)] = zero16
        ebase = w * epw

        def chunk_body(ch, _):
            off = ebase + ch * ce
            pltpu.sync_copy(pk_hbm.at[pl.ds(off, ce)], pk_v)
            pltpu.sync_copy(w_hbm.at[pl.ds(off, ce)], w_v)

            @plsc.parallel_loop(0, ce // 16, unroll=8)
            def _(i):
                pk16 = pk_v[pl.ds(i * 16, 16)]
                d16 = lax.shift_right_logical(pk16, 16)
                w16 = w_v[pl.ds(i * 16, 16)]
                plsc.addupdate_scatter(part_v, [d16], w16)
            return 0

        lax.fori_loop(0, nch, chunk_body, 0)
        pltpu.sync_copy(part_v, out_hbm.at[w])

    return deg_kernel


@functools.lru_cache(maxsize=None)
def _make_isd(npad):
    def body(parts_ref, out_ref):
        deg = jnp.sum(parts_ref[...], axis=0, keepdims=True) + 1e-6
        out_ref[...] = lax.rsqrt(deg)

    return pl.pallas_call(
        body,
        out_shape=jax.ShapeDtypeStruct((1, npad), _F32),
    )


@functools.lru_cache(maxsize=None)
def _make_wn(n):
    e = n * 16
    npad = _round_up(n, 256)
    epw = e // _NW
    ce = min(epw, 1600)
    nch = epw // ce

    @functools.partial(
        pl.kernel,
        out_type=jax.ShapeDtypeStruct((e,), _F32),
        mesh=_mesh(),
        compiler_params=_SC_PARAMS,
        scratch_types=[
            pltpu.VMEM((npad,), _F32),
            pltpu.VMEM((ce,), _I32),
            pltpu.VMEM((ce,), _F32),
            pltpu.VMEM((ce,), _F32),
        ],
    )
    def wn_kernel(pk_hbm, w_hbm, isd_hbm, out_hbm, isd_v, pk_v, w_v, o_v):
        w = _wid()
        pltpu.sync_copy(isd_hbm.at[0], isd_v)
        ebase = w * epw

        def chunk_body(ch, _):
            off = ebase + ch * ce
            pltpu.sync_copy(pk_hbm.at[pl.ds(off, ce)], pk_v)
            pltpu.sync_copy(w_hbm.at[pl.ds(off, ce)], w_v)

            @plsc.parallel_loop(0, ce // 16, unroll=8)
            def _(i):
                pk16 = pk_v[pl.ds(i * 16, 16)]
                s16 = jnp.bitwise_and(pk16, 0xFFFF)
                d16 = lax.shift_right_logical(pk16, 16)
                w16 = w_v[pl.ds(i * 16, 16)]
                a = plsc.load_gather(isd_v, [s16])
                b = plsc.load_gather(isd_v, [d16])
                o_v[pl.ds(i * 16, 16)] = w16 * a * b
            pltpu.sync_copy(o_v, out_hbm.at[pl.ds(off, ce)])
            return 0

        lax.fori_loop(0, nch, chunk_body, 0)

    return wn_kernel


@functools.lru_cache(maxsize=None)
def _make_apply(n, c_cols, cheb):
    e = n * 16
    ce = 3200
    nch = e // ce
    half = nch // 2
    budget = 112000
    ncol = max(1, min(budget // (2 * n), 32, -(-c_cols // _NW)))
    nsweep = -(-c_cols // (_NW * ncol))

    scratch = [
        pltpu.VMEM((ncol * n,), _F32),
        pltpu.VMEM((ncol * n,), _F32),
        pltpu.VMEM((2 * ce,), _I32),
        pltpu.VMEM((2 * ce,), _I32),
        pltpu.SemaphoreType.DMA,
        pltpu.SemaphoreType.DMA,
    ]

    def body(x_hbm, ew_hbm, *rest):
        if cheb:
            prev_hbm, out_hbm, x_v, a_v, e0_v, e1_v, sem0, sem1 = rest
        else:
            out_hbm, x_v, a_v, e0_v, e1_v, sem0, sem1 = rest
        w = _wid()

        unroll = max(1, min(8, 64 // max(ncol, 1)))

        def compute(buf):
            @plsc.parallel_loop(0, ce // 16, unroll=unroll)
            def _(i):
                pk16 = buf[pl.ds(i * 32, 16)]
                s16 = jnp.bitwise_and(pk16, 0xFFFF)
                d16 = lax.shift_right_logical(pk16, 16)
                w16 = plsc.bitcast(buf[pl.ds(i * 32 + 16, 16)], _F32)
                for j in range(ncol):
                    v = plsc.load_gather(x_v, [s16 + j * n])
                    plsc.addupdate_scatter(a_v, [d16 + j * n], -(v * w16))

        for sw in range(nsweep):
            base = (sw * _NW + w) * ncol

            for j in range(ncol):
                col = base + j

                @pl.when(col < c_cols)
                def _():
                    pltpu.sync_copy(x_hbm.at[col], x_v.at[pl.ds(j * n, n)])
                    pltpu.sync_copy(x_hbm.at[col], a_v.at[pl.ds(j * n, n)])

            @pl.when(base < c_cols)
            def _():
                pltpu.async_copy(ew_hbm.at[pl.ds(0, 2 * ce)], e0_v, sem0)

                def pair(p, _):
                    c0 = 2 * p
                    pltpu.async_copy(
                        ew_hbm.at[pl.ds((c0 + 1) * 2 * ce, 2 * ce)], e1_v, sem1)
                    pltpu.make_async_copy(
                        ew_hbm.at[pl.ds(c0 * 2 * ce, 2 * ce)], e0_v, sem0).wait()
                    compute(e0_v)

                    @pl.when(c0 + 2 < nch)
                    def _():
                        pltpu.async_copy(
                            ew_hbm.at[pl.ds((c0 + 2) * 2 * ce, 2 * ce)],
                            e0_v, sem0)

                    pltpu.make_async_copy(
                        ew_hbm.at[pl.ds((c0 + 1) * 2 * ce, 2 * ce)],
                        e1_v, sem1).wait()
                    compute(e1_v)
                    return 0

                lax.fori_loop(0, half, pair, 0)

            for j in range(ncol):
                col = base + j

                @pl.when(col < c_cols)
                def _():
                    if not cheb:
                        pltpu.sync_copy(a_v.at[pl.ds(j * n, n)], out_hbm.at[col])
                    else:
                        pltpu.sync_copy(prev_hbm.at[col], x_v.at[pl.ds(j * n, n)])

                        jj = j * n

                        @plsc.parallel_loop(0, n // 16, unroll=8)
                        def _(i):
                            av = a_v[pl.ds(jj + i * 16, 16)]
                            pv = x_v[pl.ds(jj + i * 16, 16)]
                            x_v[pl.ds(jj + i * 16, 16)] = 2.0 * av - pv
                        pltpu.sync_copy(x_v.at[pl.ds(j * n, n)], out_hbm.at[col])

    return functools.partial(
        pl.kernel,
        out_type=jax.ShapeDtypeStruct((c_cols, n), _F32),
        mesh=_mesh(),
        compiler_params=_SC_PARAMS,
        scratch_types=scratch,
    )(body)


@functools.lru_cache(maxsize=None)
def _make_mm(f, m, n, b, has_bias, f2, act):
    nb = min(1024, n)
    grid = (b, -(-n // nb))

    def body(*refs):
        idx = 0
        w_ref = refs[idx]; idx += 1
        t_ref = refs[idx]; idx += 1
        if has_bias:
            bias_ref = refs[idx]; idx += 1
        if f2:
            w2_ref = refs[idx]; idx += 1
            t2_ref = refs[idx]; idx += 1
        out_ref = refs[idx]
        y = lax.dot_general(
            w_ref[...], t_ref[0],
            (((0,), (0,)), ((), ())),
            precision=lax.Precision.HIGHEST,
            preferred_element_type=_F32,
        )
        if f2:
            y = y + lax.dot_general(
                w2_ref[...], t2_ref[0],
                (((0,), (0,)), ((), ())),
                precision=lax.Precision.HIGHEST,
                preferred_element_type=_F32,
            )
        if has_bias:
            y = y + bias_ref[...]
        if act == "relu":
            y = jnp.maximum(y, 0.0)
        elif act == "lsm":
            y = y - jnp.max(y, axis=0, keepdims=True)
            y = y - jnp.log(jnp.sum(jnp.exp(y), axis=0, keepdims=True))
        out_ref[0] = y

    in_specs = [
        pl.BlockSpec((f, m), lambda bi, ni: (0, 0)),
        pl.BlockSpec((1, f, nb), lambda bi, ni: (bi, 0, ni)),
    ]
    if has_bias:
        in_specs.append(pl.BlockSpec((m, 1), lambda bi, ni: (0, 0)))
    if f2:
        in_specs.append(pl.BlockSpec((f2, m), lambda bi, ni: (0, 0)))
        in_specs.append(pl.BlockSpec((1, f2, nb), lambda bi, ni: (bi, 0, ni)))

    return pl.pallas_call(
        body,
        grid=grid,
        in_specs=in_specs,
        out_specs=pl.BlockSpec((1, m, nb), lambda bi, ni: (bi, 0, ni)),
        out_shape=jax.ShapeDtypeStruct((b, m, n), _F32),
    )


@functools.lru_cache(maxsize=None)
def _make_max(r, ncols):
    br = min(r, 256)
    bn = min(ncols, 2048)
    grid = (-(-r // br), -(-ncols // bn))

    def body(a_ref, b_ref, o_ref):
        o_ref[...] = jnp.maximum(a_ref[...], b_ref[...])

    spec = pl.BlockSpec((br, bn), lambda i, j: (i, j))
    return pl.pallas_call(
        body,
        grid=grid,
        in_specs=[spec, spec],
        out_specs=spec,
        out_shape=jax.ShapeDtypeStruct((r, ncols), _F32),
    )


def _cheb_T(xbc, graph):
    ew, n = graph
    b, cin, _ = xbc.shape
    c = b * cin
    x2 = xbc.reshape(c, n)
    t1 = _make_apply(n, c, False)(x2, ew)
    t2 = _make_apply(n, c, True)(t1, ew, x2)
    return jnp.concatenate(
        [xbc, t1.reshape(b, cin, n), t2.reshape(b, cin, n)], axis=1)


def _conv_k3(xbc, p, graph, act):
    t = _cheb_T(xbc, graph)
    b, f, n = t.shape
    m = p["W"].shape[2]
    wf = p["W"].reshape(f, m)
    bias = p["b"].reshape(m, 1)
    return _make_mm(f, m, n, b, True, 0, act)(wf, t, bias)


def _res_block(xbc, p, graph):
    h = _conv_k3(xbc, p["conv1"], graph, "relu")
    t = _cheb_T(h, graph)
    b, f, n = t.shape
    cin = xbc.shape[1]
    m = p["conv2"]["W"].shape[2]
    w2f = p["conv2"]["W"].reshape(f, m)
    bias = p["conv2"]["b"].reshape(m, 1)
    wscf = p["sc"]["W"].reshape(cin, m)
    return _make_mm(f, m, n, b, True, cin, "relu")(w2f, t, bias, wscf, xbc)


def _pool(t):
    b, c, n = t.shape
    a = t[:, :, 0::2].reshape(b * c, n // 2)
    bb = t[:, :, 1::2].reshape(b * c, n // 2)
    return _make_max(b * c, n // 2)(a, bb).reshape(b, c, n // 2)


def _unpool(t):
    return jnp.repeat(t, 2, axis=2)


def kernel(x, params, edge_src, edge_dst, edge_w):
    graphs = {}
    for i, lvl in enumerate(_LVLS):
        n = _NS[i]
        src = edge_src[lvl].astype(_I32)
        dst = edge_dst[lvl].astype(_I32)
        pk = jnp.bitwise_or(src, dst << 16)
        ew = edge_w[lvl].astype(_F32)
        parts = _make_deg(n)(pk, ew)
        isd = _make_isd(_round_up(n, 256))(parts)
        wn = _make_wn(n)(pk, ew, isd)
        wn_i = lax.bitcast_convert_type(wn, _I32)
        epk = jnp.stack(
            [pk.reshape(-1, 16), wn_i.reshape(-1, 16)], axis=1).reshape(-1)
        graphs[lvl] = (epk, n)

    h = _conv_k3(x, params["enc_conv"], graphs["l5"], "relu")
    e5 = _res_block(h, params["enc_b5"], graphs["l5"])
    e4 = _res_block(_pool(e5), params["enc_b4"], graphs["l4"])
    e3 = _res_block(_pool(e4), params["enc_b3"], graphs["l3"])
    e2 = _res_block(_pool(e3), params["enc_b2"], graphs["l2"])
    e1 = _res_block(_pool(e2), params["enc_b1"], graphs["l1"])
    e0 = _res_block(_pool(e1), params["enc_b0"], graphs["l0"])
    d1 = _res_block(jnp.concatenate([_unpool(e0), e1], axis=1),
                    params["dec_b1"], graphs["l1"])
    d2 = _res_block(jnp.concatenate([_unpool(d1), e2], axis=1),
                    params["dec_b2"], graphs["l2"])
    d3 = _res_block(jnp.concatenate([_unpool(d2), e3], axis=1),
                    params["dec_b3"], graphs["l3"])
    d4 = _res_block(jnp.concatenate([_unpool(d3), e4], axis=1),
                    params["dec_b4"], graphs["l4"])
    d5 = _res_block(jnp.concatenate([_unpool(d4), e5], axis=1),
                    params["dec_b5"], graphs["l5"])

    b, cin, n = d5.shape
    wdec = params["dec_conv"]["W"].reshape(cin, 10)
    return _make_mm(cin, 10, n, b, False, 0, "lsm")(wdec, d5)


# speedup vs baseline: 111.8212x; 1.3469x over previous
import functools
import math

import jax
import jax.numpy as jnp
from jax import lax
from jax.experimental import pallas as pl
from jax.experimental.pallas import tpu as pltpu
from jax.experimental.pallas import tpu_sc as plsc

_KS = 3
_NS = [800, 1600, 3200, 6400, 12800, 25600]
_LVLS = ["l0", "l1", "l2", "l3", "l4", "l5"]
_NW = 32
_F32 = jnp.float32
_I32 = jnp.int32


def _mesh():
    return plsc.VectorSubcoreMesh(core_axis_name="c", subcore_axis_name="s")


_SC_PARAMS = pltpu.CompilerParams(
    needs_layout_passes=False, use_tc_tiling_on_sc=False)


def _wid():
    return lax.axis_index("s") * 2 + lax.axis_index("c")


def _round_up(x, m):
    return (x + m - 1) // m * m


@functools.lru_cache(maxsize=None)
def _make_deg(n):
    e = n * 16
    npad = _round_up(n, 256)
    epw = e // _NW
    ce = min(epw, 1600)
    nch = epw // ce

    @functools.partial(
        pl.kernel,
        out_type=jax.ShapeDtypeStruct((_NW, npad), _F32),
        mesh=_mesh(),
        compiler_params=_SC_PARAMS,
        scratch_types=[
            pltpu.VMEM((npad,), _F32),
            pltpu.VMEM((ce,), _I32),
            pltpu.VMEM((ce,), _F32),
        ],
    )
    def deg_kernel(pk_hbm, w_hbm, out_hbm, part_v, pk_v, w_v):
        w = _wid()
        zero16 = jnp.zeros((16,), _F32)

        @plsc.parallel_loop(0, npad // 16, unroll=8)
        def _(i):
            part_v[pl.ds(i * 16, 16)] = zero16
        ebase = w * epw

        def chunk_body(ch, _):
            off = ebase + ch * ce
            pltpu.sync_copy(pk_hbm.at[pl.ds(off, ce)], pk_v)
            pltpu.sync_copy(w_hbm.at[pl.ds(off, ce)], w_v)

            @plsc.parallel_loop(0, ce // 16, unroll=8)
            def _(i):
                pk16 = pk_v[pl.ds(i * 16, 16)]
                d16 = lax.shift_right_logical(pk16, 16)
                w16 = w_v[pl.ds(i * 16, 16)]
                plsc.addupdate_scatter(part_v, [d16], w16)
            return 0

        lax.fori_loop(0, nch, chunk_body, 0)
        pltpu.sync_copy(part_v, out_hbm.at[w])

    return deg_kernel


@functools.lru_cache(maxsize=None)
def _make_isd(npad):
    def body(parts_ref, out_ref):
        deg = jnp.sum(parts_ref[...], axis=0, keepdims=True) + 1e-6
        out_ref[...] = lax.rsqrt(deg)

    return pl.pallas_call(
        body,
        out_shape=jax.ShapeDtypeStruct((1, npad), _F32),
    )


@functools.lru_cache(maxsize=None)
def _make_wn(n):
    e = n * 16
    npad = _round_up(n, 256)
    epw = e // _NW
    ce = min(epw, 1600)
    nch = epw // ce

    @functools.partial(
        pl.kernel,
        out_type=jax.ShapeDtypeStruct((e,), _F32),
        mesh=_mesh(),
        compiler_params=_SC_PARAMS,
        scratch_types=[
            pltpu.VMEM((npad,), _F32),
            pltpu.VMEM((ce,), _I32),
            pltpu.VMEM((ce,), _F32),
            pltpu.VMEM((ce,), _F32),
        ],
    )
    def wn_kernel(pk_hbm, w_hbm, isd_hbm, out_hbm, isd_v, pk_v, w_v, o_v):
        w = _wid()
        pltpu.sync_copy(isd_hbm.at[0], isd_v)
        ebase = w * epw

        def chunk_body(ch, _):
            off = ebase + ch * ce
            pltpu.sync_copy(pk_hbm.at[pl.ds(off, ce)], pk_v)
            pltpu.sync_copy(w_hbm.at[pl.ds(off, ce)], w_v)

            @plsc.parallel_loop(0, ce // 16, unroll=8)
            def _(i):
                pk16 = pk_v[pl.ds(i * 16, 16)]
                s16 = jnp.bitwise_and(pk16, 0xFFFF)
                d16 = lax.shift_right_logical(pk16, 16)
                w16 = w_v[pl.ds(i * 16, 16)]
                a = plsc.load_gather(isd_v, [s16])
                b = plsc.load_gather(isd_v, [d16])
                o_v[pl.ds(i * 16, 16)] = w16 * a * b
            pltpu.sync_copy(o_v, out_hbm.at[pl.ds(off, ce)])
            return 0

        lax.fori_loop(0, nch, chunk_body, 0)

    return wn_kernel


@functools.lru_cache(maxsize=None)
def _make_apply(n, c_cols, cheb):
    e = n * 16
    ce = 3200
    nch = e // ce
    half = nch // 2
    budget = 112000
    ncol = max(1, min(budget // (2 * n), 32, -(-c_cols // _NW)))
    nsweep = -(-c_cols // (_NW * ncol))

    scratch = [
        pltpu.VMEM((ncol * n,), _F32),
        pltpu.VMEM((ncol * n,), _F32),
        pltpu.VMEM((2 * ce,), _I32),
        pltpu.VMEM((2 * ce,), _I32),
        pltpu.SemaphoreType.DMA,
        pltpu.SemaphoreType.DMA,
    ]

    def body(x_hbm, ew_hbm, *rest):
        if cheb:
            prev_hbm, out_hbm, x_v, a_v, e0_v, e1_v, sem0, sem1 = rest
        else:
            out_hbm, x_v, a_v, e0_v, e1_v, sem0, sem1 = rest
        w = _wid()

        unroll = max(1, min(8, 64 // max(ncol, 1)))

        def compute(buf):
            @plsc.parallel_loop(0, ce // 16, unroll=unroll)
            def _(i):
                pk16 = buf[pl.ds(i * 32, 16)]
                s16 = jnp.bitwise_and(pk16, 0xFFFF)
                d16 = lax.shift_right_logical(pk16, 16)
                w16 = plsc.bitcast(buf[pl.ds(i * 32 + 16, 16)], _F32)
                for j in range(ncol):
                    v = plsc.load_gather(x_v, [s16 + j * n])
                    plsc.addupdate_scatter(a_v, [d16 + j * n], -(v * w16))

        for sw in range(nsweep):
            base = (sw * _NW + w) * ncol

            for j in range(ncol):
                col = base + j

                @pl.when(col < c_cols)
                def _():
                    pltpu.sync_copy(x_hbm.at[col], x_v.at[pl.ds(j * n, n)])
                    pltpu.sync_copy(x_hbm.at[col], a_v.at[pl.ds(j * n, n)])

            @pl.when(base < c_cols)
            def _():
                pltpu.async_copy(ew_hbm.at[pl.ds(0, 2 * ce)], e0_v, sem0)

                def pair(p, _):
                    c0 = 2 * p
                    pltpu.async_copy(
                        ew_hbm.at[pl.ds((c0 + 1) * 2 * ce, 2 * ce)], e1_v, sem1)
                    pltpu.make_async_copy(
                        ew_hbm.at[pl.ds(c0 * 2 * ce, 2 * ce)], e0_v, sem0).wait()
                    compute(e0_v)

                    @pl.when(c0 + 2 < nch)
                    def _():
                        pltpu.async_copy(
                            ew_hbm.at[pl.ds((c0 + 2) * 2 * ce, 2 * ce)],
                            e0_v, sem0)

                    pltpu.make_async_copy(
                        ew_hbm.at[pl.ds((c0 + 1) * 2 * ce, 2 * ce)],
                        e1_v, sem1).wait()
                    compute(e1_v)
                    return 0

                lax.fori_loop(0, half, pair, 0)

            for j in range(ncol):
                col = base + j

                @pl.when(col < c_cols)
                def _():
                    if not cheb:
                        pltpu.sync_copy(a_v.at[pl.ds(j * n, n)], out_hbm.at[col])
                    else:
                        pltpu.sync_copy(prev_hbm.at[col], x_v.at[pl.ds(j * n, n)])

                        jj = j * n

                        @plsc.parallel_loop(0, n // 16, unroll=8)
                        def _(i):
                            av = a_v[pl.ds(jj + i * 16, 16)]
                            pv = x_v[pl.ds(jj + i * 16, 16)]
                            x_v[pl.ds(jj + i * 16, 16)] = 2.0 * av - pv
                        pltpu.sync_copy(x_v.at[pl.ds(j * n, n)], out_hbm.at[col])

    return functools.partial(
        pl.kernel,
        out_type=jax.ShapeDtypeStruct((c_cols, n), _F32),
        mesh=_mesh(),
        compiler_params=_SC_PARAMS,
        scratch_types=scratch,
    )(body)


@functools.lru_cache(maxsize=None)
def _make_dec_apply(n, c_cols):
    e = n * 16
    ce = 3200
    nch = e // ce
    half = nch // 2
    budget = 112000
    ncol = max(1, min(budget // (2 * n), 32, -(-c_cols // _NW)))
    nsweep = -(-c_cols // (_NW * ncol))

    scratch = [
        pltpu.VMEM((ncol * n,), _F32),
        pltpu.VMEM((ncol * n,), _F32),
        pltpu.VMEM((2 * ce,), _I32),
        pltpu.VMEM((2 * ce,), _I32),
        pltpu.SemaphoreType.DMA,
        pltpu.SemaphoreType.DMA,
    ]

    @functools.partial(
        pl.kernel,
        out_type=jax.ShapeDtypeStruct((c_cols, n), _F32),
        mesh=_mesh(),
        compiler_params=_SC_PARAMS,
        scratch_types=scratch,
    )
    def dec_kernel(z2_hbm, z1_hbm, base_hbm, ew_hbm, out_hbm,
                   x_v, a_v, e0_v, e1_v, sem0, sem1):
        w = _wid()
        unroll = max(1, min(8, 64 // max(ncol, 1)))

        def compute(buf):
            @plsc.parallel_loop(0, ce // 16, unroll=unroll)
            def _(i):
                pk16 = buf[pl.ds(i * 32, 16)]
                s16 = jnp.bitwise_and(pk16, 0xFFFF)
                d16 = lax.shift_right_logical(pk16, 16)
                w16 = plsc.bitcast(buf[pl.ds(i * 32 + 16, 16)], _F32)
                for j in range(ncol):
                    v = plsc.load_gather(x_v, [s16 + j * n])
                    plsc.addupdate_scatter(a_v, [d16 + j * n], -(v * w16))

        def edge_pass():
            pltpu.async_copy(ew_hbm.at[pl.ds(0, 2 * ce)], e0_v, sem0)

            def pair(p, _):
                c0 = 2 * p
                pltpu.async_copy(
                    ew_hbm.at[pl.ds((c0 + 1) * 2 * ce, 2 * ce)], e1_v, sem1)
                pltpu.make_async_copy(
                    ew_hbm.at[pl.ds(c0 * 2 * ce, 2 * ce)], e0_v, sem0).wait()
                compute(e0_v)

                @pl.when(c0 + 2 < nch)
                def _():
                    pltpu.async_copy(
                        ew_hbm.at[pl.ds((c0 + 2) * 2 * ce, 2 * ce)],
                        e0_v, sem0)

                pltpu.make_async_copy(
                    ew_hbm.at[pl.ds((c0 + 1) * 2 * ce, 2 * ce)],
                    e1_v, sem1).wait()
                compute(e1_v)
                return 0

            lax.fori_loop(0, half, pair, 0)

        for sw in range(nsweep):
            base = (sw * _NW + w) * ncol

            for j in range(ncol):
                col = base + j

                @pl.when(col < c_cols)
                def _():
                    pltpu.sync_copy(z2_hbm.at[col], x_v.at[pl.ds(j * n, n)])
                    pltpu.sync_copy(z2_hbm.at[col], a_v.at[pl.ds(j * n, n)])

            @pl.when(base < c_cols)
            def _():
                edge_pass()

            for j in range(ncol):
                col = base + j

                @pl.when(col < c_cols)
                def _():
                    pltpu.sync_copy(z1_hbm.at[col], x_v.at[pl.ds(j * n, n)])

            @pl.when(base < c_cols)
            def _():
                @plsc.parallel_loop(0, ncol * n // 16, unroll=8)
                def _(i):
                    t = x_v[pl.ds(i * 16, 16)] + 2.0 * a_v[pl.ds(i * 16, 16)]
                    x_v[pl.ds(i * 16, 16)] = t
                    a_v[pl.ds(i * 16, 16)] = t

                edge_pass()

            for j in range(ncol):
                col = base + j
                jj = j * n

                @pl.when(col < c_cols)
                def _():
                    pltpu.sync_copy(base_hbm.at[col], x_v.at[pl.ds(jj, n)])

                    @plsc.parallel_loop(0, n // 16, unroll=8)
                    def _(i):
                        t = x_v[pl.ds(jj + i * 16, 16)] + a_v[pl.ds(jj + i * 16, 16)]
                        x_v[pl.ds(jj + i * 16, 16)] = jnp.maximum(t, 0.0)

                    pltpu.sync_copy(x_v.at[pl.ds(jj, n)], out_hbm.at[col])

    return dec_kernel


@functools.lru_cache(maxsize=None)
def _make_mm(f, m, n, b, has_bias, f2, act):
    nb = min(1024, n)
    grid = (b, -(-n // nb))

    def body(*refs):
        idx = 0
        w_ref = refs[idx]; idx += 1
        t_ref = refs[idx]; idx += 1
        if has_bias:
            bias_ref = refs[idx]; idx += 1
        if f2:
            w2_ref = refs[idx]; idx += 1
            t2_ref = refs[idx]; idx += 1
        out_ref = refs[idx]
        y = lax.dot_general(
            w_ref[...], t_ref[0],
            (((0,), (0,)), ((), ())),
            precision=lax.Precision.HIGHEST,
            preferred_element_type=_F32,
        )
        if f2:
            y = y + lax.dot_general(
                w2_ref[...], t2_ref[0],
                (((0,), (0,)), ((), ())),
                precision=lax.Precision.HIGHEST,
                preferred_element_type=_F32,
            )
        if has_bias:
            y = y + bias_ref[...]
        if act == "relu":
            y = jnp.maximum(y, 0.0)
        elif act == "lsm":
            y = y - jnp.max(y, axis=0, keepdims=True)
            y = y - jnp.log(jnp.sum(jnp.exp(y), axis=0, keepdims=True))
        out_ref[0] = y

    in_specs = [
        pl.BlockSpec((f, m), lambda bi, ni: (0, 0)),
        pl.BlockSpec((1, f, nb), lambda bi, ni: (bi, 0, ni)),
    ]
    if has_bias:
        in_specs.append(pl.BlockSpec((m, 1), lambda bi, ni: (0, 0)))
    if f2:
        in_specs.append(pl.BlockSpec((f2, m), lambda bi, ni: (0, 0)))
        in_specs.append(pl.BlockSpec((1, f2, nb), lambda bi, ni: (bi, 0, ni)))

    return pl.pallas_call(
        body,
        grid=grid,
        in_specs=in_specs,
        out_specs=pl.BlockSpec((1, m, nb), lambda bi, ni: (bi, 0, ni)),
        out_shape=jax.ShapeDtypeStruct((b, m, n), _F32),
    )


@functools.lru_cache(maxsize=None)
def _make_max(r, ncols):
    br = min(r, 256)
    bn = min(ncols, 2048)
    grid = (-(-r // br), -(-ncols // bn))

    def body(a_ref, b_ref, o_ref):
        o_ref[...] = jnp.maximum(a_ref[...], b_ref[...])

    spec = pl.BlockSpec((br, bn), lambda i, j: (i, j))
    return pl.pallas_call(
        body,
        grid=grid,
        in_specs=[spec, spec],
        out_specs=spec,
        out_shape=jax.ShapeDtypeStruct((r, ncols), _F32),
    )


def _cheb_T(xbc, graph):
    ew, n = graph
    b, cin, _ = xbc.shape
    c = b * cin
    x2 = xbc.reshape(c, n)
    t1 = _make_apply(n, c, False)(x2, ew)
    t2 = _make_apply(n, c, True)(t1, ew, x2)
    return jnp.concatenate(
        [xbc, t1.reshape(b, cin, n), t2.reshape(b, cin, n)], axis=1)


def _conv_k3(xbc, p, graph, act):
    t = _cheb_T(xbc, graph)
    b, f, n = t.shape
    m = p["W"].shape[2]
    wf = p["W"].reshape(f, m)
    bias = p["b"].reshape(m, 1)
    return _make_mm(f, m, n, b, True, 0, act)(wf, t, bias)


def _conv_k3_commuted(xbc, p, graph):
    ew, n = graph
    b, cin, _ = xbc.shape
    cout = p["W"].shape[2]
    w0, w1, w2 = p["W"][0], p["W"][1], p["W"][2]
    wp = jnp.concatenate([w0 - w2, w1, w2], axis=1)
    bias3 = jnp.concatenate(
        [p["b"], jnp.zeros((2 * cout,), _F32)]).reshape(3 * cout, 1)
    y = _make_mm(cin, 3 * cout, n, b, True, 0, "none")(wp, xbc, bias3)
    c = b * cout
    base = y[:, :cout].reshape(c, n)
    z1 = y[:, cout:2 * cout].reshape(c, n)
    z2 = y[:, 2 * cout:].reshape(c, n)
    out = _make_dec_apply(n, c)(z2, z1, base, ew)
    return out.reshape(b, cout, n)


def _res_block(xbc, p, graph):
    if p["conv1"]["W"].shape[2] < xbc.shape[1]:
        h = _conv_k3_commuted(xbc, p["conv1"], graph)
    else:
        h = _conv_k3(xbc, p["conv1"], graph, "relu")
    t = _cheb_T(h, graph)
    b, f, n = t.shape
    cin = xbc.shape[1]
    m = p["conv2"]["W"].shape[2]
    w2f = p["conv2"]["W"].reshape(f, m)
    bias = p["conv2"]["b"].reshape(m, 1)
    wscf = p["sc"]["W"].reshape(cin, m)
    return _make_mm(f, m, n, b, True, cin, "relu")(w2f, t, bias, wscf, xbc)


def _pool(t):
    b, c, n = t.shape
    a = t[:, :, 0::2].reshape(b * c, n // 2)
    bb = t[:, :, 1::2].reshape(b * c, n // 2)
    return _make_max(b * c, n // 2)(a, bb).reshape(b, c, n // 2)


def _unpool(t):
    return jnp.repeat(t, 2, axis=2)


def kernel(x, params, edge_src, edge_dst, edge_w):
    graphs = {}
    for i, lvl in enumerate(_LVLS):
        n = _NS[i]
        src = edge_src[lvl].astype(_I32)
        dst = edge_dst[lvl].astype(_I32)
        pk = jnp.bitwise_or(src, dst << 16)
        ew = edge_w[lvl].astype(_F32)
        parts = _make_deg(n)(pk, ew)
        isd = _make_isd(_round_up(n, 256))(parts)
        wn = _make_wn(n)(pk, ew, isd)
        wn_i = lax.bitcast_convert_type(wn, _I32)
        epk = jnp.stack(
            [pk.reshape(-1, 16), wn_i.reshape(-1, 16)], axis=1).reshape(-1)
        graphs[lvl] = (epk, n)

    h = _conv_k3(x, params["enc_conv"], graphs["l5"], "relu")
    e5 = _res_block(h, params["enc_b5"], graphs["l5"])
    e4 = _res_block(_pool(e5), params["enc_b4"], graphs["l4"])
    e3 = _res_block(_pool(e4), params["enc_b3"], graphs["l3"])
    e2 = _res_block(_pool(e3), params["enc_b2"], graphs["l2"])
    e1 = _res_block(_pool(e2), params["enc_b1"], graphs["l1"])
    e0 = _res_block(_pool(e1), params["enc_b0"], graphs["l0"])
    d1 = _res_block(jnp.concatenate([_unpool(e0), e1], axis=1),
                    params["dec_b1"], graphs["l1"])
    d2 = _res_block(jnp.concatenate([_unpool(d1), e2], axis=1),
                    params["dec_b2"], graphs["l2"])
    d3 = _res_block(jnp.concatenate([_unpool(d2), e3], axis=1),
                    params["dec_b3"], graphs["l3"])
    d4 = _res_block(jnp.concatenate([_unpool(d3), e4], axis=1),
                    params["dec_b4"], graphs["l4"])
    d5 = _res_block(jnp.concatenate([_unpool(d4), e5], axis=1),
                    params["dec_b5"], graphs["l5"])

    b, cin, n = d5.shape
    wdec = params["dec_conv"]["W"].reshape(cin, 10)
    return _make_mm(cin, 10, n, b, False, 0, "lsm")(wdec, d5)
